# baseline (speedup 1.0000x reference)
import jax
import jax.numpy as jnp
from jax import lax
from jax.experimental import pallas as pl
from jax.experimental.pallas import tpu as pltpu
from jax.experimental.pallas import tpu_sc as plsc

B = 16384
D = 32
K = 20
NC, NS, L = 2, 16, 16
NW = NC * NS
BPW = B // NW
V = 1000000
WIN = 512
NFULL = 1953
HG = 64
NHG = BPW // HG


def _conv_body(out_t, emb_t, tail_o, tail_e, conv_o, conv_e,
               winA, winB, tpA, tpB,
               semIA, semIB, semOA, semOB):
    wid = lax.axis_index("s") * NC + lax.axis_index("c")
    iota = lax.iota(jnp.int32, L)
    iota32 = iota * D
    for (src, tail16, conv) in ((out_t, tail_o, conv_o),
                                (emb_t, tail_e, conv_e)):
        _one_table(src, tail16, conv, wid, iota32,
                   winA, winB, tpA, tpB, semIA, semIB, semOA, semOB)


def _one_table(out_t, tail16, conv, wid, iota32,
               winA, winB, tpA, tpB, semIA, semIB, semOA, semOB):

    def fire_in(t, buf, sem):
        for dt in range(4):
            pltpu.async_copy(out_t.at[pl.ds(dt * 8, 8), pl.ds(t * WIN, WIN)],
                             buf.at[pl.ds(dt * 8, 8), pl.ds(0, WIN)], sem)

    def drain_in(buf, sem):
        for dt in range(4):
            pltpu.make_async_copy(
                out_t.at[pl.ds(0, 8), pl.ds(0, WIN)],
                buf.at[pl.ds(dt * 8, 8), pl.ds(0, WIN)], sem).wait()

    def transpose(buf, tp):
        iota_l = lax.iota(jnp.int32, L)

        def kstep(kk, _):
            for u in range(8):
                k = kk * 8 + u
                g0 = plsc.load_gather(buf, [iota_l, jnp.full((L,), k,
                                                            jnp.int32)])
                g1 = plsc.load_gather(buf, [iota_l + L, jnp.full((L,), k,
                                                                jnp.int32)])
                tp[pl.ds(k * D, L)] = g0
                tp[pl.ds(k * D + L, L)] = g1
            return _
        lax.fori_loop(0, WIN // 8, kstep, 0)

    def fire_out(t, tp, sem):
        pltpu.async_copy(tp, conv.at[pl.ds(t * (WIN * D), WIN * D)], sem)

    def drain_out(tp, sem):
        pltpu.make_async_copy(conv.at[pl.ds(0, WIN * D)], tp, sem).wait()

    def tA(jj):
        return wid + 32 * (2 * jj)

    def tB(jj):
        return wid + 32 * (2 * jj + 1)

    fire_in(tA(0), winA, semIA)
    fire_in(tB(0), winB, semIB)

    def step(jj, cr):
        validB = jnp.logical_or(jj < 30, wid == 0)

        drain_in(winA, semIA)

        @pl.when(jj > 0)
        def _():
            drain_out(tpA, semOA)
        transpose(winA, tpA)
        fire_out(tA(jj), tpA, semOA)

        @pl.when(jj < 30)
        def _():
            fire_in(tA(jj + 1), winA, semIA)

        @pl.when(validB)
        def _():
            drain_in(winB, semIB)

            @pl.when(jj > 0)
            def _():
                drain_out(tpB, semOB)
            transpose(winB, tpB)
            fire_out(tB(jj), tpB, semOB)

        @pl.when(jnp.logical_or(jj < 29,
                                jnp.logical_and(jj == 29, wid == 0)))
        def _():
            fire_in(tB(jj + 1), winB, semIB)
        return cr
    lax.fori_loop(0, 31, step, 0)

    drain_out(tpA, semOA)

    @pl.when(wid == 0)
    def _():
        drain_out(tpB, semOB)

    @pl.when(wid == 0)
    def _():
        def trow(r, cr2):
            pltpu.sync_copy(tail16.at[r],
                            conv.at[pl.ds(NFULL * WIN * D + r * 128, 128)])
            return cr2
        lax.fori_loop(0, 16, trow, 0)


def _convert(out_t, emb_t, tail_o, tail_e):
    mesh = plsc.VectorSubcoreMesh(core_axis_name="c", subcore_axis_name="s")
    f = pl.kernel(
        _conv_body,
        out_type=[
            jax.ShapeDtypeStruct((V * D,), jnp.float32),
            jax.ShapeDtypeStruct((V * D,), jnp.float32),
        ],
        mesh=mesh,
        scratch_types=[
            pltpu.VMEM((D, WIN + 8), jnp.float32),
            pltpu.VMEM((D, WIN + 8), jnp.float32),
            pltpu.VMEM((WIN * D,), jnp.float32),
            pltpu.VMEM((WIN * D,), jnp.float32),
            pltpu.SemaphoreType.DMA,
            pltpu.SemaphoreType.DMA,
            pltpu.SemaphoreType.DMA,
            pltpu.SemaphoreType.DMA,
        ],
        compiler_params=pltpu.CompilerParams(needs_layout_passes=False,
                                             use_tc_tiling_on_sc=True),
    )
    return f(out_t, emb_t, tail_o, tail_e)


def _tc_t_body(x_ref, y_ref, o_ref, p_ref):
    x = x_ref[...]
    y = y_ref[...]
    r = lax.broadcasted_iota(jnp.int32, (4 * D, WIN), 0)
    kk = lax.broadcasted_iota(jnp.int32, (4 * D, WIN), 1)
    d1 = lax.broadcasted_iota(jnp.int32, (D, 4 * D), 0)
    c1 = lax.broadcasted_iota(jnp.int32, (D, 4 * D), 1)
    acc_o = jnp.zeros((4 * D, 4 * D), jnp.float32)
    acc_p = jnp.zeros((4 * D, 4 * D), jnp.float32)
    for q in range(4):
        S = (kk == 4 * r + q).astype(jnp.float32)
        E = (c1 == q * D + d1).astype(jnp.float32)
        yo = lax.dot_general(S, x, (((1,), (1,)), ((), ())),
                             preferred_element_type=jnp.float32)
        yp = lax.dot_general(S, y, (((1,), (1,)), ((), ())),
                             preferred_element_type=jnp.float32)
        acc_o = acc_o + jnp.dot(yo, E, preferred_element_type=jnp.float32)
        acc_p = acc_p + jnp.dot(yp, E, preferred_element_type=jnp.float32)
    o_ref[...] = acc_o
    p_ref[...] = acc_p


def _tc_transpose(out_t, emb_t):
    grid = (V + WIN - 1) // WIN
    return pl.pallas_call(
        _tc_t_body,
        grid=(grid,),
        in_specs=[pl.BlockSpec((D, WIN), lambda t: (0, t)),
                  pl.BlockSpec((D, WIN), lambda t: (0, t))],
        out_specs=[pl.BlockSpec((4 * D, 4 * D), lambda t: (t, 0)),
                   pl.BlockSpec((4 * D, 4 * D), lambda t: (t, 0))],
        out_shape=[jax.ShapeDtypeStruct((V // 4, 4 * D), jnp.float32),
                   jax.ShapeDtypeStruct((V // 4, 4 * D), jnp.float32)],
    )(out_t, emb_t)


def _gather_body(conv, cemb, uvb, negsb, pos_hbm, neg_hbm,
                 idxuA, idxuB, idxnA, idxnB, u_bufA, u_bufB,
                 v_bufA, v_bufB, vpA, vpB, posA, posB, negA, negB,
                 semXA, semXB, semGA, semGB, semWA, semWB):
    wid = lax.axis_index("s") * NC + lax.axis_index("c")
    iota = lax.iota(jnp.int32, L)

    def gidx(h):
        return wid * NHG + h

    def fire_idx(h, idxuv, idxn, sem):
        pltpu.async_copy(uvb.at[gidx(h)], idxuv, sem)
        pltpu.async_copy(negsb.at[gidx(h)], idxn, sem)

    def drain_idx(idxuv, idxn, sem):
        pltpu.make_async_copy(uvb.at[0], idxuv, sem).wait()
        pltpu.make_async_copy(negsb.at[0], idxn, sem).wait()

    def fire_g(idxuv, idxn, u_buf, v_buf, vp, sem):
        pltpu.async_copy(cemb.at[idxuv.at[pl.ds(0, HG)]], u_buf, sem)
        pltpu.async_copy(conv.at[idxuv.at[pl.ds(HG, HG)]], v_buf, sem)
        for jh in range(K // 2):
            pltpu.async_copy(conv.at[idxn.at[jh]], vp.at[jh], sem)

    def drain_g(u_buf, v_buf, vp, sem):
        pltpu.make_async_copy(cemb.at[idxuA.at[pl.ds(0, HG)]],
                              u_buf, sem).wait()
        pltpu.make_async_copy(cemb.at[idxuA.at[pl.ds(0, HG)]],
                              v_buf, sem).wait()
        for jh in range(K // 2):
            pltpu.make_async_copy(cemb.at[idxuA.at[pl.ds(0, HG)]],
                                  vp.at[jh], sem).wait()

    lane_hi = iota == (L - 1)

    def compute(u_buf, v_buf, vp, pos_buf, neg_buf):
        def item(k, _):
            u0 = u_buf[k, pl.ds(0, L)]
            u1 = u_buf[k, pl.ds(L, L)]
            v0 = v_buf[k, pl.ds(0, L)]
            v1 = v_buf[k, pl.ds(L, L)]
            sp = plsc.cumsum(u0 * v0 + u1 * v1)
            plsc.store_scatter(pos_buf, [jnp.full((L,), k, jnp.int32)],
                               sp, mask=lane_hi)
            for j in range(K):
                w0 = vp[j // 2, (j % 2) * HG + k, pl.ds(0, L)]
                w1 = vp[j // 2, (j % 2) * HG + k, pl.ds(L, L)]
                sj = plsc.cumsum(u0 * w0 + u1 * w1)
                plsc.store_scatter(neg_buf,
                                   [jnp.full((L,), j, jnp.int32),
                                    jnp.full((L,), k, jnp.int32)],
                                   sj, mask=lane_hi)
            return _
        lax.fori_loop(0, HG, item, 0)

    def fire_out(h, pos_buf, neg_buf, sem):
        pltpu.async_copy(pos_buf, pos_hbm.at[pl.ds(gidx(h) * HG, HG)], sem)
        pltpu.async_copy(neg_buf, neg_hbm.at[gidx(h)], sem)

    def drain_out(pos_buf, neg_buf, sem):
        pltpu.make_async_copy(pos_buf, pos_hbm.at[pl.ds(0, HG)], sem).wait()
        pltpu.make_async_copy(neg_buf, neg_hbm.at[0], sem).wait()

    fire_idx(0, idxuA, idxnA, semXA)
    drain_idx(idxuA, idxnA, semXA)
    fire_g(idxuA, idxnA, u_bufA, v_bufA, vpA, semGA)
    fire_idx(1, idxuB, idxnB, semXB)

    def step(jj, cr):
        hA = 2 * jj
        hB = 2 * jj + 1
        drain_idx(idxuB, idxnB, semXB)
        fire_g(idxuB, idxnB, u_bufB, v_bufB, vpB, semGB)

        drain_g(u_bufA, v_bufA, vpA, semGA)
        compute(u_bufA, v_bufA, vpA, posA, negA)

        @pl.when(jj > 0)
        def _():
            drain_out(posA, negA, semWA)
        fire_out(hA, posA, negA, semWA)

        @pl.when(jj < 3)
        def _():
            fire_idx(hA + 2, idxuA, idxnA, semXA)
            drain_idx(idxuA, idxnA, semXA)
            fire_g(idxuA, idxnA, u_bufA, v_bufA, vpA, semGA)
            fire_idx(hB + 2, idxuB, idxnB, semXB)

        drain_g(u_bufB, v_bufB, vpB, semGB)
        compute(u_bufB, v_bufB, vpB, posB, negB)

        @pl.when(jj > 0)
        def _():
            drain_out(posB, negB, semWB)
        fire_out(hB, posB, negB, semWB)
        return cr
    lax.fori_loop(0, NHG // 2, step, 0)

    drain_out(posA, negA, semWA)
    drain_out(posB, negB, semWB)


def _gather(conv, cemb, uvb, negsb):
    mesh = plsc.VectorSubcoreMesh(core_axis_name="c", subcore_axis_name="s")
    f = pl.kernel(
        _gather_body,
        out_type=[
            jax.ShapeDtypeStruct((B,), jnp.float32),
            jax.ShapeDtypeStruct((NW * NHG, K, HG), jnp.float32),
        ],
        mesh=mesh,
        scratch_types=[
            pltpu.VMEM((2 * HG,), jnp.int32),
            pltpu.VMEM((2 * HG,), jnp.int32),
            pltpu.VMEM((K // 2, 2 * HG), jnp.int32),
            pltpu.VMEM((K // 2, 2 * HG), jnp.int32),
            pltpu.VMEM((HG, D), jnp.float32),
            pltpu.VMEM((HG, D), jnp.float32),
            pltpu.VMEM((HG, D), jnp.float32),
            pltpu.VMEM((HG, D), jnp.float32),
            pltpu.VMEM((K // 2, 2 * HG, D), jnp.float32),
            pltpu.VMEM((K // 2, 2 * HG, D), jnp.float32),
            pltpu.VMEM((HG,), jnp.float32),
            pltpu.VMEM((HG,), jnp.float32),
            pltpu.VMEM((K, HG), jnp.float32),
            pltpu.VMEM((K, HG), jnp.float32),
            pltpu.SemaphoreType.DMA,
            pltpu.SemaphoreType.DMA,
            pltpu.SemaphoreType.DMA,
            pltpu.SemaphoreType.DMA,
            pltpu.SemaphoreType.DMA,
            pltpu.SemaphoreType.DMA,
        ],
        compiler_params=pltpu.CompilerParams(needs_layout_passes=False,
                                             use_tc_tiling_on_sc=False),
    )
    return f(conv, cemb, uvb, negsb)


def _tc_loss_body(pos_ref, neg_ref, out_ref):
    p = pos_ref[...]
    n = neg_ref[...]

    def logsig(x):
        return jnp.minimum(x, 0.0) - jnp.log1p(jnp.exp(-jnp.abs(x)))

    total = jnp.sum(logsig(p)) + jnp.sum(logsig(-n))
    out_ref[...] = jnp.reshape(-total / B, (1, 1))


def _tc_loss(pos, neg):
    return pl.pallas_call(
        _tc_loss_body,
        out_shape=jax.ShapeDtypeStruct((1, 1), jnp.float32),
    )(pos.reshape(128, B // 128), neg.reshape(B * K // 128, 128))


def kernel(context, target, negatives, emb_table, out_table):
    out_t = out_table.T
    emb_t = emb_table.T
    neg_t = negatives.astype(jnp.int32).T
    tail_o = out_table[NFULL * WIN:, :].reshape(16, 128)
    tail_e = emb_table[NFULL * WIN:, :].reshape(16, 128)
    conv, cemb = _convert(out_t, emb_t, tail_o, tail_e)
    uvb = jnp.concatenate([context.astype(jnp.int32).reshape(B // HG, HG),
                           target.astype(jnp.int32).reshape(B // HG, HG)],
                          axis=1)
    negsb = (neg_t.reshape(K, B // HG, HG)
             .transpose(1, 0, 2).reshape(B // HG, K // 2, 2 * HG))
    pos_sims, neg_sims = _gather(conv.reshape(V, D), cemb.reshape(V, D),
                                 uvb, negsb)
    loss = _tc_loss(pos_sims, neg_sims.reshape(-1))
    return loss[0, 0]


# speedup vs baseline: 1.5076x; 1.5076x over previous
import jax
import jax.numpy as jnp
from jax import lax
from jax.experimental import pallas as pl
from jax.experimental.pallas import tpu as pltpu
from jax.experimental.pallas import tpu_sc as plsc

B = 16384
D = 32
K = 20
NC, NS, L = 2, 16, 16
NW = NC * NS
BPW = B // NW
V = 1000000
WIN = 512
NFULL = 1953
HG = 64
NHG = BPW // HG


def _conv_body(out_t, emb_t, tail_o, tail_e, conv_o, conv_e,
               winA, winB, tpA, tpB,
               semIA, semIB, semOA, semOB):
    wid = lax.axis_index("s") * NC + lax.axis_index("c")
    iota = lax.iota(jnp.int32, L)
    iota32 = iota * D
    for (src, tail16, conv) in ((out_t, tail_o, conv_o),
                                (emb_t, tail_e, conv_e)):
        _one_table(src, tail16, conv, wid, iota32,
                   winA, winB, tpA, tpB, semIA, semIB, semOA, semOB)


def _one_table(out_t, tail16, conv, wid, iota32,
               winA, winB, tpA, tpB, semIA, semIB, semOA, semOB):

    def fire_in(t, buf, sem):
        for dt in range(4):
            pltpu.async_copy(out_t.at[pl.ds(dt * 8, 8), pl.ds(t * WIN, WIN)],
                             buf.at[pl.ds(dt * 8, 8), pl.ds(0, WIN)], sem)

    def drain_in(buf, sem):
        for dt in range(4):
            pltpu.make_async_copy(
                out_t.at[pl.ds(0, 8), pl.ds(0, WIN)],
                buf.at[pl.ds(dt * 8, 8), pl.ds(0, WIN)], sem).wait()

    def transpose(buf, tp):
        iota_l = lax.iota(jnp.int32, L)

        def kstep(kk, _):
            for u in range(8):
                k = kk * 8 + u
                g0 = plsc.load_gather(buf, [iota_l, jnp.full((L,), k,
                                                            jnp.int32)])
                g1 = plsc.load_gather(buf, [iota_l + L, jnp.full((L,), k,
                                                                jnp.int32)])
                tp[pl.ds(k * D, L)] = g0
                tp[pl.ds(k * D + L, L)] = g1
            return _
        lax.fori_loop(0, WIN // 8, kstep, 0)

    def fire_out(t, tp, sem):
        pltpu.async_copy(tp, conv.at[pl.ds(t * (WIN * D), WIN * D)], sem)

    def drain_out(tp, sem):
        pltpu.make_async_copy(conv.at[pl.ds(0, WIN * D)], tp, sem).wait()

    def tA(jj):
        return wid + 32 * (2 * jj)

    def tB(jj):
        return wid + 32 * (2 * jj + 1)

    fire_in(tA(0), winA, semIA)
    fire_in(tB(0), winB, semIB)

    def step(jj, cr):
        validB = jnp.logical_or(jj < 30, wid == 0)

        drain_in(winA, semIA)

        @pl.when(jj > 0)
        def _():
            drain_out(tpA, semOA)
        transpose(winA, tpA)
        fire_out(tA(jj), tpA, semOA)

        @pl.when(jj < 30)
        def _():
            fire_in(tA(jj + 1), winA, semIA)

        @pl.when(validB)
        def _():
            drain_in(winB, semIB)

            @pl.when(jj > 0)
            def _():
                drain_out(tpB, semOB)
            transpose(winB, tpB)
            fire_out(tB(jj), tpB, semOB)

        @pl.when(jnp.logical_or(jj < 29,
                                jnp.logical_and(jj == 29, wid == 0)))
        def _():
            fire_in(tB(jj + 1), winB, semIB)
        return cr
    lax.fori_loop(0, 31, step, 0)

    drain_out(tpA, semOA)

    @pl.when(wid == 0)
    def _():
        drain_out(tpB, semOB)

    @pl.when(wid == 0)
    def _():
        def trow(r, cr2):
            pltpu.sync_copy(tail16.at[r],
                            conv.at[pl.ds(NFULL * WIN * D + r * 128, 128)])
            return cr2
        lax.fori_loop(0, 16, trow, 0)


def _convert(out_t, emb_t, tail_o, tail_e):
    mesh = plsc.VectorSubcoreMesh(core_axis_name="c", subcore_axis_name="s")
    f = pl.kernel(
        _conv_body,
        out_type=[
            jax.ShapeDtypeStruct((V * D,), jnp.float32),
            jax.ShapeDtypeStruct((V * D,), jnp.float32),
        ],
        mesh=mesh,
        scratch_types=[
            pltpu.VMEM((D, WIN + 8), jnp.float32),
            pltpu.VMEM((D, WIN + 8), jnp.float32),
            pltpu.VMEM((WIN * D,), jnp.float32),
            pltpu.VMEM((WIN * D,), jnp.float32),
            pltpu.SemaphoreType.DMA,
            pltpu.SemaphoreType.DMA,
            pltpu.SemaphoreType.DMA,
            pltpu.SemaphoreType.DMA,
        ],
        compiler_params=pltpu.CompilerParams(needs_layout_passes=False,
                                             use_tc_tiling_on_sc=True),
    )
    return f(out_t, emb_t, tail_o, tail_e)


def _tc_t_body(x_ref, y_ref, o_ref, p_ref):
    x = x_ref[...]
    y = y_ref[...]
    r = lax.broadcasted_iota(jnp.int32, (4 * D, WIN), 0)
    kk = lax.broadcasted_iota(jnp.int32, (4 * D, WIN), 1)
    d1 = lax.broadcasted_iota(jnp.int32, (D, 4 * D), 0)
    c1 = lax.broadcasted_iota(jnp.int32, (D, 4 * D), 1)
    acc_o = jnp.zeros((4 * D, 4 * D), jnp.float32)
    acc_p = jnp.zeros((4 * D, 4 * D), jnp.float32)
    for q in range(4):
        S = (kk == 4 * r + q).astype(jnp.float32)
        E = (c1 == q * D + d1).astype(jnp.float32)
        yo = lax.dot_general(S, x, (((1,), (1,)), ((), ())),
                             preferred_element_type=jnp.float32)
        yp = lax.dot_general(S, y, (((1,), (1,)), ((), ())),
                             preferred_element_type=jnp.float32)
        acc_o = acc_o + jnp.dot(yo, E, preferred_element_type=jnp.float32)
        acc_p = acc_p + jnp.dot(yp, E, preferred_element_type=jnp.float32)
    o_ref[...] = acc_o
    p_ref[...] = acc_p


def _tc_transpose(out_t, emb_t):
    grid = (V + WIN - 1) // WIN
    return pl.pallas_call(
        _tc_t_body,
        grid=(grid,),
        in_specs=[pl.BlockSpec((D, WIN), lambda t: (0, t)),
                  pl.BlockSpec((D, WIN), lambda t: (0, t))],
        out_specs=[pl.BlockSpec((4 * D, 4 * D), lambda t: (t, 0)),
                   pl.BlockSpec((4 * D, 4 * D), lambda t: (t, 0))],
        out_shape=[jax.ShapeDtypeStruct((V // 4, 4 * D), jnp.float32),
                   jax.ShapeDtypeStruct((V // 4, 4 * D), jnp.float32)],
    )(out_t, emb_t)


def _gather_body(conv, cemb, uvb, negsb, pos_hbm, neg_hbm,
                 idxuA, idxuB, idxnA, idxnB, u_bufA, u_bufB,
                 v_bufA, v_bufB, vpA, vpB, posA, posB, negA, negB,
                 semXA, semXB, semGA, semGB, semWA, semWB):
    wid = lax.axis_index("s") * NC + lax.axis_index("c")
    iota = lax.iota(jnp.int32, L)

    def gidx(h):
        return wid * NHG + h

    def fire_idx(h, idxuv, idxn, sem):
        pltpu.async_copy(uvb.at[gidx(h)], idxuv, sem)
        pltpu.async_copy(negsb.at[gidx(h)], idxn, sem)

    def drain_idx(idxuv, idxn, sem):
        pltpu.make_async_copy(uvb.at[0], idxuv, sem).wait()
        pltpu.make_async_copy(negsb.at[0], idxn, sem).wait()

    def fire_g(idxuv, idxn, u_buf, v_buf, vp, sem):
        pltpu.async_copy(cemb.at[idxuv.at[pl.ds(0, HG)]], u_buf, sem)
        pltpu.async_copy(conv.at[idxuv.at[pl.ds(HG, HG)]], v_buf, sem)
        for jh in range(K // 2):
            pltpu.async_copy(conv.at[idxn.at[jh]], vp.at[jh], sem)

    def drain_g(u_buf, v_buf, vp, sem):
        pltpu.make_async_copy(cemb.at[idxuA.at[pl.ds(0, HG)]],
                              u_buf, sem).wait()
        pltpu.make_async_copy(cemb.at[idxuA.at[pl.ds(0, HG)]],
                              v_buf, sem).wait()
        for jh in range(K // 2):
            pltpu.make_async_copy(cemb.at[idxuA.at[pl.ds(0, HG)]],
                                  vp.at[jh], sem).wait()

    lane_hi = iota == (L - 1)

    def compute(u_buf, v_buf, vp, pos_buf, neg_buf):
        def item(k, _):
            u0 = u_buf[k, pl.ds(0, L)]
            u1 = u_buf[k, pl.ds(L, L)]
            v0 = v_buf[k, pl.ds(0, L)]
            v1 = v_buf[k, pl.ds(L, L)]
            sp = plsc.cumsum(u0 * v0 + u1 * v1)
            plsc.store_scatter(pos_buf, [jnp.full((L,), k, jnp.int32)],
                               sp, mask=lane_hi)
            for j in range(K):
                w0 = vp[j // 2, (j % 2) * HG + k, pl.ds(0, L)]
                w1 = vp[j // 2, (j % 2) * HG + k, pl.ds(L, L)]
                sj = plsc.cumsum(u0 * w0 + u1 * w1)
                plsc.store_scatter(neg_buf,
                                   [jnp.full((L,), j, jnp.int32),
                                    jnp.full((L,), k, jnp.int32)],
                                   sj, mask=lane_hi)
            return _
        lax.fori_loop(0, HG, item, 0)

    def fire_out(h, pos_buf, neg_buf, sem):
        pltpu.async_copy(pos_buf, pos_hbm.at[pl.ds(gidx(h) * HG, HG)], sem)
        pltpu.async_copy(neg_buf, neg_hbm.at[gidx(h)], sem)

    def drain_out(pos_buf, neg_buf, sem):
        pltpu.make_async_copy(pos_buf, pos_hbm.at[pl.ds(0, HG)], sem).wait()
        pltpu.make_async_copy(neg_buf, neg_hbm.at[0], sem).wait()

    fire_idx(0, idxuA, idxnA, semXA)
    drain_idx(idxuA, idxnA, semXA)
    fire_g(idxuA, idxnA, u_bufA, v_bufA, vpA, semGA)
    fire_idx(1, idxuB, idxnB, semXB)

    def step(jj, cr):
        hA = 2 * jj
        hB = 2 * jj + 1
        drain_idx(idxuB, idxnB, semXB)
        fire_g(idxuB, idxnB, u_bufB, v_bufB, vpB, semGB)

        drain_g(u_bufA, v_bufA, vpA, semGA)
        compute(u_bufA, v_bufA, vpA, posA, negA)

        @pl.when(jj > 0)
        def _():
            drain_out(posA, negA, semWA)
        fire_out(hA, posA, negA, semWA)

        @pl.when(jj < 3)
        def _():
            fire_idx(hA + 2, idxuA, idxnA, semXA)
            drain_idx(idxuA, idxnA, semXA)
            fire_g(idxuA, idxnA, u_bufA, v_bufA, vpA, semGA)
            fire_idx(hB + 2, idxuB, idxnB, semXB)

        drain_g(u_bufB, v_bufB, vpB, semGB)
        compute(u_bufB, v_bufB, vpB, posB, negB)

        @pl.when(jj > 0)
        def _():
            drain_out(posB, negB, semWB)
        fire_out(hB, posB, negB, semWB)
        return cr
    lax.fori_loop(0, NHG // 2, step, 0)

    drain_out(posA, negA, semWA)
    drain_out(posB, negB, semWB)


def _gather(conv, cemb, uvb, negsb):
    mesh = plsc.VectorSubcoreMesh(core_axis_name="c", subcore_axis_name="s")
    f = pl.kernel(
        _gather_body,
        out_type=[
            jax.ShapeDtypeStruct((B,), jnp.float32),
            jax.ShapeDtypeStruct((NW * NHG, K, HG), jnp.float32),
        ],
        mesh=mesh,
        scratch_types=[
            pltpu.VMEM((2 * HG,), jnp.int32),
            pltpu.VMEM((2 * HG,), jnp.int32),
            pltpu.VMEM((K // 2, 2 * HG), jnp.int32),
            pltpu.VMEM((K // 2, 2 * HG), jnp.int32),
            pltpu.VMEM((HG, D), jnp.float32),
            pltpu.VMEM((HG, D), jnp.float32),
            pltpu.VMEM((HG, D), jnp.float32),
            pltpu.VMEM((HG, D), jnp.float32),
            pltpu.VMEM((K // 2, 2 * HG, D), jnp.float32),
            pltpu.VMEM((K // 2, 2 * HG, D), jnp.float32),
            pltpu.VMEM((HG,), jnp.float32),
            pltpu.VMEM((HG,), jnp.float32),
            pltpu.VMEM((K, HG), jnp.float32),
            pltpu.VMEM((K, HG), jnp.float32),
            pltpu.SemaphoreType.DMA,
            pltpu.SemaphoreType.DMA,
            pltpu.SemaphoreType.DMA,
            pltpu.SemaphoreType.DMA,
            pltpu.SemaphoreType.DMA,
            pltpu.SemaphoreType.DMA,
        ],
        compiler_params=pltpu.CompilerParams(needs_layout_passes=False,
                                             use_tc_tiling_on_sc=False),
    )
    return f(conv, cemb, uvb, negsb)


def _tc_loss_body(pos_ref, neg_ref, out_ref):
    p = pos_ref[...]
    n = neg_ref[...]

    def logsig(x):
        return jnp.minimum(x, 0.0) - jnp.log1p(jnp.exp(-jnp.abs(x)))

    total = jnp.sum(logsig(p)) + jnp.sum(logsig(-n))
    out_ref[...] = jnp.reshape(-total / B, (1, 1))


def _tc_loss(pos, neg):
    return pl.pallas_call(
        _tc_loss_body,
        out_shape=jax.ShapeDtypeStruct((1, 1), jnp.float32),
    )(pos.reshape(128, B // 128), neg.reshape(B * K // 128, 128))


def kernel(context, target, negatives, emb_table, out_table):
    neg_t = negatives.astype(jnp.int32).T
    conv = out_table
    cemb = emb_table
    uvb = jnp.concatenate([context.astype(jnp.int32).reshape(B // HG, HG),
                           target.astype(jnp.int32).reshape(B // HG, HG)],
                          axis=1)
    negsb = (neg_t.reshape(K, B // HG, HG)
             .transpose(1, 0, 2).reshape(B // HG, K // 2, 2 * HG))
    pos_sims, neg_sims = _gather(conv, cemb, uvb, negsb)
    loss = _tc_loss(pos_sims, neg_sims.reshape(-1))
    return loss[0, 0]
